# baseline (speedup 1.0000x reference)
import functools

import jax
import jax.numpy as jnp
from jax import lax
from jax.experimental import pallas as pl
from jax.experimental.pallas import tpu as pltpu
from jax.experimental.pallas import tpu_sc as plsc

B, C, H, W, K = 32, 16, 128, 128, 500
HW = H * W
L = 16
KPAD = 512
NCHUNK = KPAD // L


def _build_sc_kernel():
    mesh = plsc.VectorSubcoreMesh(core_axis_name="c", subcore_axis_name="s")
    nc = 2

    @functools.partial(
        pl.kernel,
        mesh=mesh,
        compiler_params=pltpu.CompilerParams(needs_layout_passes=False),
        out_type=[
            jax.ShapeDtypeStruct((B, L), jnp.float32),
            jax.ShapeDtypeStruct((B, L), jnp.float32),
        ],
        scratch_types=[
            pltpu.VMEM((KPAD,), jnp.int32),
            pltpu.VMEM((KPAD * C,), jnp.float32),
            pltpu.VMEM((KPAD,), jnp.float32),
            pltpu.VMEM((HW,), jnp.float32),
            pltpu.VMEM((L,), jnp.float32),
            pltpu.VMEM((L,), jnp.float32),
        ],
    )
    def sc_kernel(feat_hbm, ind_hbm, tgt_hbm, msk_hbm, loss_out, mask_out,
                  ind_v, tgt_v, msk_v, plane_v, lstage, mstage):
        wid = lax.axis_index("s") * nc + lax.axis_index("c")
        lanes = lax.iota(jnp.int32, L)

        pltpu.sync_copy(ind_hbm.at[wid], ind_v)
        pltpu.sync_copy(tgt_hbm.at[wid], tgt_v)
        pltpu.sync_copy(msk_hbm.at[wid], msk_v)

        def mask_body(j, macc):
            return macc + msk_v[pl.ds(j * L, L)]

        mstage[...] = lax.fori_loop(0, NCHUNK, mask_body,
                                    jnp.zeros((L,), jnp.float32))
        pltpu.sync_copy(mstage, mask_out.at[wid])

        def plane_pass(c, acc):
            def body(j, a):
                hw = ind_v[pl.ds(j * L, L)]
                pred = plsc.load_gather(plane_v, [hw])
                tidx = lanes * C + (j * (L * C) + c)
                tgt = plsc.load_gather(tgt_v, [tidx])
                d = jnp.abs(pred - tgt)
                valid = (j * L + lanes) < K
                return a + jnp.where(valid, d, 0.0)

            return lax.fori_loop(0, NCHUNK, body, acc)

        acc = jnp.zeros((L,), jnp.float32)
        for c in range(C):
            pltpu.sync_copy(feat_hbm.at[wid * C + c], plane_v)
            acc = plane_pass(c, acc)

        lstage[...] = acc
        pltpu.sync_copy(lstage, loss_out.at[wid])

    return sc_kernel


def kernel(output, ind, target, reg_mask):
    feat = output.reshape(B * C, HW)
    ind_p = jnp.zeros((B, KPAD), jnp.int32).at[:, :K].set(ind.astype(jnp.int32))
    tgt_p = (jnp.zeros((B, KPAD * C), jnp.float32)
             .at[:, : K * C].set(target.reshape(B, K * C)))
    msk_p = jnp.zeros((B, KPAD), jnp.float32).at[:, :K].set(reg_mask)
    loss_parts, mask_parts = _build_sc_kernel()(feat, ind_p, tgt_p, msk_p)
    return jnp.sum(loss_parts) / (jnp.sum(mask_parts) + 0.0001)


# speedup vs baseline: 1.1263x; 1.1263x over previous
import functools

import jax
import jax.numpy as jnp
from jax import lax
from jax.experimental import pallas as pl
from jax.experimental.pallas import tpu as pltpu
from jax.experimental.pallas import tpu_sc as plsc

B, C, H, W, K = 32, 16, 128, 128, 500
HW = H * W
L = 16
NFULL = K // L
KREM = K - NFULL * L


def _build_sc_kernel():
    mesh = plsc.VectorSubcoreMesh(core_axis_name="c", subcore_axis_name="s")
    nc = 2

    @functools.partial(
        pl.kernel,
        mesh=mesh,
        compiler_params=pltpu.CompilerParams(needs_layout_passes=False),
        out_type=[
            jax.ShapeDtypeStruct((B, L), jnp.float32),
            jax.ShapeDtypeStruct((B, L), jnp.float32),
        ],
        scratch_types=[
            pltpu.VMEM((K,), jnp.int32),
            pltpu.VMEM((K * C,), jnp.float32),
            pltpu.VMEM((K,), jnp.float32),
            pltpu.VMEM((HW,), jnp.float32),
            pltpu.VMEM((HW,), jnp.float32),
            pltpu.VMEM((L,), jnp.float32),
            pltpu.VMEM((L,), jnp.float32),
            pltpu.SemaphoreType.DMA,
            pltpu.SemaphoreType.DMA,
        ],
    )
    def sc_kernel(feat_hbm, ind_hbm, tgt_hbm, msk_hbm, loss_out, mask_out,
                  ind_v, tgt_v, msk_v, plane_a, plane_b, lstage, mstage,
                  sem_a, sem_b):
        wid = lax.axis_index("s") * nc + lax.axis_index("c")
        lanes = lax.iota(jnp.int32, L)
        tail = lanes < KREM

        planes = (plane_a, plane_b)
        sems = (sem_a, sem_b)
        cps = [None, None]
        cps[0] = pltpu.async_copy(feat_hbm.at[wid * C], plane_a, sem_a)

        pltpu.sync_copy(ind_hbm.at[wid], ind_v)
        pltpu.sync_copy(tgt_hbm.at[wid], tgt_v)
        pltpu.sync_copy(msk_hbm.at[wid], msk_v)

        def mask_body(j, macc):
            return macc + msk_v[pl.ds(j * L, L)]

        macc = lax.fori_loop(0, NFULL, mask_body, jnp.zeros((L,), jnp.float32))
        mtail = plsc.load_gather(msk_v, [NFULL * L + lanes], mask=tail)
        mstage[...] = macc + jnp.where(tail, mtail, 0.0)
        pltpu.sync_copy(mstage, mask_out.at[wid])

        def plane_pass(c, plane_v, acc):
            def body(j, a):
                hw = ind_v[pl.ds(j * L, L)]
                pred = plsc.load_gather(plane_v, [hw])
                tgt = plsc.load_gather(tgt_v, [lanes * C + (j * (L * C) + c)])
                return a + jnp.abs(pred - tgt)

            acc = lax.fori_loop(0, NFULL, body, acc)
            hw = plsc.load_gather(ind_v, [NFULL * L + lanes], mask=tail)
            pred = plsc.load_gather(plane_v, [hw], mask=tail)
            tgt = plsc.load_gather(
                tgt_v, [lanes * C + (NFULL * (L * C) + c)], mask=tail)
            return acc + jnp.where(tail, jnp.abs(pred - tgt), 0.0)

        acc = jnp.zeros((L,), jnp.float32)
        for c in range(C):
            cur = c % 2
            if c + 1 < C:
                cps[1 - cur] = pltpu.async_copy(
                    feat_hbm.at[wid * C + c + 1], planes[1 - cur], sems[1 - cur])
            cps[cur].wait()
            acc = plane_pass(c, planes[cur], acc)

        lstage[...] = acc
        pltpu.sync_copy(lstage, loss_out.at[wid])

    return sc_kernel


def kernel(output, ind, target, reg_mask):
    feat = output.reshape(B * C, HW)
    tgt = target.reshape(B, K * C)
    loss_parts, mask_parts = _build_sc_kernel()(
        feat, ind.astype(jnp.int32), tgt, reg_mask)
    return jnp.sum(loss_parts) / (jnp.sum(mask_parts) + 0.0001)


# speedup vs baseline: 1.7557x; 1.5588x over previous
import functools

import jax
import jax.numpy as jnp
from jax import lax
from jax.experimental import pallas as pl
from jax.experimental.pallas import tpu as pltpu
from jax.experimental.pallas import tpu_sc as plsc

B, C, H, W, K = 32, 16, 128, 128, 500
HW = H * W
L = 16
NFULL = K // L
KREM = K - NFULL * L


def _build_sc_kernel():
    mesh = plsc.VectorSubcoreMesh(core_axis_name="c", subcore_axis_name="s")
    nc = 2

    @functools.partial(
        pl.kernel,
        mesh=mesh,
        compiler_params=pltpu.CompilerParams(needs_layout_passes=False),
        out_type=[
            jax.ShapeDtypeStruct((B, L), jnp.float32),
            jax.ShapeDtypeStruct((B, L), jnp.float32),
        ],
        scratch_types=[
            pltpu.VMEM((K,), jnp.int32),
            pltpu.VMEM((K * C,), jnp.float32),
            pltpu.VMEM((K,), jnp.float32),
            pltpu.VMEM((H, W), jnp.float32),
            pltpu.VMEM((H, W), jnp.float32),
            pltpu.VMEM((L,), jnp.float32),
            pltpu.VMEM((L,), jnp.float32),
            pltpu.SemaphoreType.DMA,
            pltpu.SemaphoreType.DMA,
        ],
    )
    def sc_kernel(feat_hbm, ind_hbm, tgt_hbm, msk_hbm, loss_out, mask_out,
                  ind_v, tgt_v, msk_v, plane_a, plane_b, lstage, mstage,
                  sem_a, sem_b):
        wid = lax.axis_index("s") * nc + lax.axis_index("c")
        lanes = lax.iota(jnp.int32, L)
        tail = lanes < KREM

        planes = (plane_a, plane_b)
        sems = (sem_a, sem_b)
        cps = [None, None]
        cps[0] = pltpu.async_copy(feat_hbm.at[wid, 0], plane_a, sem_a)

        pltpu.sync_copy(ind_hbm.at[wid], ind_v)
        pltpu.sync_copy(tgt_hbm.at[wid], tgt_v)
        pltpu.sync_copy(msk_hbm.at[wid], msk_v)

        def mask_body(j, macc):
            return macc + msk_v[pl.ds(j * L, L)]

        macc = lax.fori_loop(0, NFULL, mask_body, jnp.zeros((L,), jnp.float32))
        mtail = plsc.load_gather(msk_v, [NFULL * L + lanes], mask=tail)
        mstage[...] = macc + jnp.where(tail, mtail, 0.0)
        pltpu.sync_copy(mstage, mask_out.at[wid])

        def plane_pass(c, plane_v, acc):
            def body(j, a):
                hw = ind_v[pl.ds(j * L, L)]
                h = lax.shift_right_logical(hw, 7)
                w = lax.bitwise_and(hw, 127)
                pred = plsc.load_gather(plane_v, [h, w])
                tgt = plsc.load_gather(tgt_v, [lanes * C + (j * (L * C) + c)])
                return a + jnp.abs(pred - tgt)

            acc = lax.fori_loop(0, NFULL, body, acc)
            hw = plsc.load_gather(ind_v, [NFULL * L + lanes], mask=tail)
            h = lax.shift_right_logical(hw, 7)
            w = lax.bitwise_and(hw, 127)
            pred = plsc.load_gather(plane_v, [h, w], mask=tail)
            tgt = plsc.load_gather(
                tgt_v, [lanes * C + (NFULL * (L * C) + c)], mask=tail)
            return acc + jnp.where(tail, jnp.abs(pred - tgt), 0.0)

        acc = jnp.zeros((L,), jnp.float32)
        for c in range(C):
            cur = c % 2
            if c + 1 < C:
                cps[1 - cur] = pltpu.async_copy(
                    feat_hbm.at[wid, c + 1], planes[1 - cur], sems[1 - cur])
            cps[cur].wait()
            acc = plane_pass(c, planes[cur], acc)

        lstage[...] = acc
        pltpu.sync_copy(lstage, loss_out.at[wid])

    return sc_kernel


def kernel(output, ind, target, reg_mask):
    tgt = target.reshape(B, K * C)
    loss_parts, mask_parts = _build_sc_kernel()(
        output, ind.astype(jnp.int32), tgt, reg_mask)
    return jnp.sum(loss_parts) / (jnp.sum(mask_parts) + 0.0001)


# speedup vs baseline: 1.7907x; 1.0199x over previous
import functools

import jax
import jax.numpy as jnp
from jax import lax
from jax.experimental import pallas as pl
from jax.experimental.pallas import tpu as pltpu
from jax.experimental.pallas import tpu_sc as plsc

B, C, H, W, K = 32, 16, 128, 128, 500
HW = H * W
L = 16
KPAD = 512
NCHUNK = KPAD // L
NDMA = KPAD // 128
NFULL = K // L
KREM = K - NFULL * L


def _build_sc_kernel():
    mesh = plsc.VectorSubcoreMesh(core_axis_name="c", subcore_axis_name="s")
    nc = 2

    @functools.partial(
        pl.kernel,
        mesh=mesh,
        compiler_params=pltpu.CompilerParams(needs_layout_passes=False),
        out_type=[
            jax.ShapeDtypeStruct((B, L), jnp.float32),
            jax.ShapeDtypeStruct((B, L), jnp.float32),
        ],
        scratch_types=[
            pltpu.VMEM((K,), jnp.int32),
            pltpu.VMEM((K * C,), jnp.float32),
            pltpu.VMEM((K,), jnp.float32),
            pltpu.VMEM((C * KPAD,), jnp.int32),
            pltpu.VMEM((C * KPAD,), jnp.float32),
            pltpu.VMEM((L,), jnp.float32),
            pltpu.VMEM((L,), jnp.float32),
            pltpu.SemaphoreType.DMA,
            pltpu.SemaphoreType.DMA,
        ],
    )
    def sc_kernel(feat_hbm, ind_hbm, tgt_hbm, msk_hbm, loss_out, mask_out,
                  ind_v, tgt_v, msk_v, idx_v, pred_v, lstage, mstage,
                  sem_a, sem_b):
        wid = lax.axis_index("s") * nc + lax.axis_index("c")
        lanes = lax.iota(jnp.int32, L)
        tail = lanes < KREM

        pltpu.sync_copy(ind_hbm.at[wid], ind_v)
        pltpu.sync_copy(tgt_hbm.at[wid], tgt_v)
        pltpu.sync_copy(msk_hbm.at[wid], msk_v)

        def mask_body(j, macc):
            return macc + msk_v[pl.ds(j * L, L)]

        macc = lax.fori_loop(0, NFULL, mask_body, jnp.zeros((L,), jnp.float32))
        mtail = plsc.load_gather(msk_v, [NFULL * L + lanes], mask=tail)
        mstage[...] = macc + jnp.where(tail, mtail, 0.0)
        pltpu.sync_copy(mstage, mask_out.at[wid])

        hw_tail_raw = plsc.load_gather(ind_v, [NFULL * L + lanes], mask=tail)
        hw_tail = jnp.where(tail, hw_tail_raw, 0)

        for c in range(C):
            base = (wid * C + c) * HW

            def build_body(g, _, c=c, base=base):
                hw = ind_v[pl.ds(g * L, L)]
                idx_v[pl.ds(c * KPAD + g * L, L)] = hw + base
                return 0

            lax.fori_loop(0, NFULL, build_body, 0)
            idx_v[pl.ds(c * KPAD + NFULL * L, L)] = hw_tail + base

        sems = (sem_a, sem_b)
        cps = [None, None]

        def fire(c):
            return [
                pltpu.async_copy(
                    feat_hbm.at[idx_v.at[pl.ds(c * KPAD + r * 128, 128)]],
                    pred_v.at[pl.ds(c * KPAD + r * 128, 128)],
                    sems[c % 2])
                for r in range(NDMA)
            ]

        def accumulate(c, acc):
            def body(g, a):
                pred = pred_v[pl.ds(c * KPAD + g * L, L)]
                tgt = plsc.load_gather(tgt_v, [lanes * C + (g * (L * C) + c)])
                return a + jnp.abs(pred - tgt)

            acc = lax.fori_loop(0, NFULL, body, acc)
            pred = pred_v[pl.ds(c * KPAD + NFULL * L, L)]
            tgt = plsc.load_gather(
                tgt_v, [lanes * C + (NFULL * (L * C) + c)], mask=tail)
            return acc + jnp.where(tail, jnp.abs(pred - tgt), 0.0)

        acc = jnp.zeros((L,), jnp.float32)
        cps[0] = fire(0)
        for c in range(C):
            if c + 1 < C:
                cps[(c + 1) % 2] = fire(c + 1)
            for d in cps[c % 2]:
                d.wait()
            acc = accumulate(c, acc)

        lstage[...] = acc
        pltpu.sync_copy(lstage, loss_out.at[wid])

    return sc_kernel


def kernel(output, ind, target, reg_mask):
    feat = output.reshape(B * C * HW)
    tgt = target.reshape(B, K * C)
    loss_parts, mask_parts = _build_sc_kernel()(
        feat, ind.astype(jnp.int32), tgt, reg_mask)
    return jnp.sum(loss_parts) / (jnp.sum(mask_parts) + 0.0001)


# speedup vs baseline: 1.9174x; 1.0708x over previous
import functools

import jax
import jax.numpy as jnp
from jax import lax
from jax.experimental import pallas as pl
from jax.experimental.pallas import tpu as pltpu
from jax.experimental.pallas import tpu_sc as plsc

B, C, H, W, K = 32, 16, 128, 128, 500
HW = H * W
L = 16
KPAD = 512
NFULL = K // L
KREM = K - NFULL * L
NGRP = 8
GCH = KPAD // NGRP
DPG = GCH * L // 128


def _build_sc_kernel():
    mesh = plsc.VectorSubcoreMesh(core_axis_name="c", subcore_axis_name="s")
    nc = 2

    @functools.partial(
        pl.kernel,
        mesh=mesh,
        compiler_params=pltpu.CompilerParams(needs_layout_passes=False),
        out_type=[
            jax.ShapeDtypeStruct((B, L), jnp.float32),
            jax.ShapeDtypeStruct((B, L), jnp.float32),
        ],
        scratch_types=[
            pltpu.VMEM((K,), jnp.int32),
            pltpu.VMEM((K * C,), jnp.float32),
            pltpu.VMEM((K,), jnp.float32),
            pltpu.VMEM((KPAD * C,), jnp.int32),
            pltpu.VMEM((KPAD * C,), jnp.float32),
            pltpu.VMEM((L,), jnp.float32),
            pltpu.VMEM((L,), jnp.float32),
            pltpu.SemaphoreType.DMA,
            pltpu.SemaphoreType.DMA,
        ],
    )
    def sc_kernel(feat_hbm, ind_hbm, tgt_hbm, msk_hbm, loss_out, mask_out,
                  ind_v, tgt_v, msk_v, idx_v, pred_v, lstage, mstage,
                  sem_a, sem_b):
        wid = lax.axis_index("s") * nc + lax.axis_index("c")
        lanes = lax.iota(jnp.int32, L)
        tail = lanes < KREM
        chan_base = (wid * C + lanes) * HW

        pltpu.sync_copy(ind_hbm.at[wid], ind_v)
        pltpu.sync_copy(tgt_hbm.at[wid], tgt_v)
        pltpu.sync_copy(msk_hbm.at[wid], msk_v)

        def mask_body(j, macc):
            return macc + msk_v[pl.ds(j * L, L)]

        macc = lax.fori_loop(0, NFULL, mask_body, jnp.zeros((L,), jnp.float32))
        mtail = plsc.load_gather(msk_v, [NFULL * L + lanes], mask=tail)
        mstage[...] = macc + jnp.where(tail, mtail, 0.0)
        pltpu.sync_copy(mstage, mask_out.at[wid])

        def build_body(t, _):
            tv = jnp.full((L,), t, jnp.int32)
            hw = plsc.load_gather(ind_v, [tv])
            idx_v[pl.ds(t * L, L)] = chan_base + hw
            return 0

        def build_pad_body(t, _):
            idx_v[pl.ds(t * L, L)] = chan_base
            return 0

        def build(g):
            if g < NGRP - 1:
                lax.fori_loop(g * GCH, (g + 1) * GCH, build_body, 0)
            else:
                lax.fori_loop(g * GCH, K, build_body, 0)
                lax.fori_loop(K, KPAD, build_pad_body, 0)

        sems = (sem_a, sem_b)

        def fire(g):
            return [
                pltpu.async_copy(
                    feat_hbm.at[idx_v.at[pl.ds((g * DPG + r) * 128, 128)]],
                    pred_v.at[pl.ds((g * DPG + r) * 128, 128)],
                    sems[g % 2])
                for r in range(DPG)
            ]

        def accumulate(g, acc):
            def body(t, a):
                pred = pred_v[pl.ds(t * L, L)]
                tgt = tgt_v[pl.ds(t * L, L)]
                return a + jnp.abs(pred - tgt)

            return lax.fori_loop(g * GCH, min((g + 1) * GCH, K), body, acc)

        cps = [None, None]
        build(0)
        cps[0] = fire(0)
        acc = jnp.zeros((L,), jnp.float32)
        for g in range(NGRP):
            if g + 1 < NGRP:
                build(g + 1)
                cps[(g + 1) % 2] = fire(g + 1)
            for d in cps[g % 2]:
                d.wait()
            acc = accumulate(g, acc)

        lstage[...] = acc
        pltpu.sync_copy(lstage, loss_out.at[wid])

    return sc_kernel


def kernel(output, ind, target, reg_mask):
    feat = output.reshape(B * C * HW)
    tgt = target.reshape(B, K * C)
    loss_parts, mask_parts = _build_sc_kernel()(
        feat, ind.astype(jnp.int32), tgt, reg_mask)
    return jnp.sum(loss_parts) / (jnp.sum(mask_parts) + 0.0001)
